# baseline (speedup 1.0000x reference)
import functools

import jax
import jax.numpy as jnp
from jax import lax
from jax.experimental import pallas as pl
from jax.experimental.pallas import tpu as pltpu
from jax.experimental.pallas import tpu_sc as plsc

_NC = 2
_NS = 16
_L = 16
_NW = _NC * _NS
_B = 256
_K = 192
_OFFLEN = 272
_BIG = 0x7FFFFFFF


def _sc_segment_reduce(x, batch, G):
    N, C = x.shape
    CG = C // _L
    segs_per_w = _B // _NW
    OV = _OFFLEN // _L

    mesh = plsc.VectorSubcoreMesh(
        core_axis_name="c", subcore_axis_name="s",
        num_cores=_NC, num_subcores=_NS)

    @functools.partial(
        pl.kernel,
        out_type=(
            jax.ShapeDtypeStruct((_B, C), jnp.float32),
            jax.ShapeDtypeStruct((_B, C), jnp.float32),
            jax.ShapeDtypeStruct((_OFFLEN,), jnp.int32),
        ),
        mesh=mesh,
        scratch_types=[
            pltpu.VMEM((G + _L,), jnp.int32),
            pltpu.VMEM((_OFFLEN,), jnp.int32),
            pltpu.VMEM((_NS, _OFFLEN), jnp.int32),
            pltpu.VMEM_SHARED((_NS, _OFFLEN), jnp.int32),
            pltpu.VMEM((2, _K, C), jnp.float32),
            pltpu.VMEM((segs_per_w, C), jnp.float32),
            pltpu.VMEM((segs_per_w, C), jnp.float32),
            pltpu.SemaphoreType.DMA((2,)),
        ],
        compiler_params=pltpu.CompilerParams(needs_layout_passes=False),
    )
    def sc_kernel(x_hbm, b_hbm, sum_hbm, max_hbm, off_hbm,
                  idsbuf, offloc, comb, shared, buf_v, res_s, res_m, sem):
        t = lax.axis_index("s")
        w = t * _NC + lax.axis_index("c")

        @pl.when(t == 0)
        def _():
            idsbuf[pl.ds(0, _L)] = jnp.full((_L,), -1, jnp.int32)
            pltpu.sync_copy(b_hbm.at[pl.ds(0, G + 8)],
                            idsbuf.at[pl.ds(8, G + 8)])

        @pl.when(jnp.logical_and(t > 0, t < _NS - 1))
        def _():
            pltpu.sync_copy(b_hbm.at[pl.ds(t * G - 8, G + _L)], idsbuf)

        @pl.when(t == _NS - 1)
        def _():
            pltpu.sync_copy(b_hbm.at[pl.ds((_NS - 1) * G - 8,
                                           N - (_NS - 1) * G + 8)],
                            idsbuf.at[pl.ds(0, N - (_NS - 1) * G + 8)])

        big = jnp.full((_L,), _BIG, jnp.int32)
        for v in range(OV):
            offloc[pl.ds(v * _L, _L)] = big

        iota = lax.iota(jnp.int32, _L)
        nv_last = -(-(N - (_NS - 1) * G) // _L)
        nvec = jnp.where(t == _NS - 1, nv_last, G // _L)

        def scan_body(k, _):
            i = _L * k
            cur = idsbuf[pl.ds(i + 8, _L)]
            prv = idsbuf[pl.ds(i + 7, _L)]
            pos = jnp.broadcast_to(t * G + i, (_L,)) + iota
            msk = jnp.logical_and(cur != prv, pos < N)
            plsc.store_scatter(offloc, [cur], pos, mask=msk)
            return 0

        lax.fori_loop(0, nvec, scan_body, 0)

        pltpu.sync_copy(offloc, shared.at[t])
        plsc.subcore_barrier()
        pltpu.sync_copy(shared, comb)

        carry = jnp.int32(_BIG)
        for v in reversed(range(OV)):
            mvec = comb[0, pl.ds(v * _L, _L)]
            for tt in range(1, _NS):
                mvec = jnp.minimum(mvec, comb[tt, pl.ds(v * _L, _L)])
            if v == OV - 1:
                mvec = jnp.minimum(mvec, jnp.full((_L,), N, jnp.int32))
            sm = -lax.rev(plsc.cummax(lax.rev(-mvec, (0,))), (0,))
            tot = jnp.minimum(sm, jnp.broadcast_to(carry, (_L,)))
            offloc[pl.ds(v * _L, _L)] = tot
            carry = tot[0]

        @pl.when(w == 0)
        def _():
            pltpu.sync_copy(offloc, off_hbm)

        offv = offloc[pl.ds(w * segs_per_w, _L)]

        W0 = offv[0]
        W8 = offv[segs_per_w]
        base0 = pl.multiple_of((W0 // 8) * 8, 8)
        nblk = jnp.where(W8 > W0, (W8 - base0 + (_K - 1)) // _K, 0)

        def dma_for(blk, slot):
            base = base0 + blk * _K
            cb = pl.multiple_of(jnp.minimum(base, N - _K), 8)
            return pltpu.make_async_copy(
                x_hbm.at[pl.ds(cb, _K)], buf_v.at[slot], sem.at[slot])

        @pl.when(nblk > 0)
        def _():
            dma_for(0, 0).start()

        def acc_init():
            return (
                tuple(jnp.zeros((_L,), jnp.float32) for _ in range(CG)),
                tuple(jnp.full((_L,), -jnp.inf, jnp.float32) for _ in range(CG)),
            )

        zero = jnp.zeros((_L,), jnp.float32)

        def blk_body(blk, carry):
            p = lax.rem(blk, 2)
            dma_for(blk, p).wait()

            @pl.when(blk + 1 < nblk)
            def _():
                dma_for(blk + 1, 1 - p).start()

            base = base0 + blk * _K
            cb = pl.multiple_of(jnp.minimum(base, N - _K), 8)

            for s in range(segs_per_w):
                s_start = offv[s]
                s_end = offv[s + 1]
                lo = jnp.maximum(s_start, base) - cb
                hi = jnp.maximum(jnp.minimum(s_end, base + _K) - cb, lo)
                n2 = (hi - lo) // 2

                def pair_body(i, c, lo=lo, p=p):
                    sums, maxs = c
                    r = lo + 2 * i
                    new_s = []
                    new_m = []
                    for cg in range(CG):
                        v1 = buf_v[p, r, pl.ds(cg * _L, _L)]
                        v2 = buf_v[p, r + 1, pl.ds(cg * _L, _L)]
                        new_s.append(sums[cg] + (v1 + v2))
                        new_m.append(jnp.maximum(maxs[cg],
                                                 jnp.maximum(v1, v2)))
                    return (tuple(new_s), tuple(new_m))

                def row_body(r, c, p=p):
                    sums, maxs = c
                    new_s = []
                    new_m = []
                    for cg in range(CG):
                        v = buf_v[p, r, pl.ds(cg * _L, _L)]
                        new_s.append(sums[cg] + v)
                        new_m.append(jnp.maximum(maxs[cg], v))
                    return (tuple(new_s), tuple(new_m))

                carry = lax.fori_loop(0, n2, pair_body, carry)
                carry = lax.fori_loop(lo + 2 * n2, hi, row_body, carry)

                completes = jnp.logical_and(
                    s_end <= base + _K,
                    jnp.logical_or(s_end > base, blk == 0))

                def flush(c, s=s, s_start=s_start, s_end=s_end):
                    sums, maxs = c
                    for cg in range(CG):
                        res_s[s, pl.ds(cg * _L, _L)] = sums[cg]

                    @pl.when(s_end > s_start)
                    def _():
                        for cg in range(CG):
                            res_m[s, pl.ds(cg * _L, _L)] = maxs[cg]

                    @pl.when(s_end == s_start)
                    def _():
                        for cg in range(CG):
                            res_m[s, pl.ds(cg * _L, _L)] = zero

                    return acc_init()

                carry = lax.cond(completes, flush, lambda c: c, carry)

            return carry

        lax.fori_loop(0, nblk, blk_body, acc_init())

        @pl.when(nblk == 0)
        def _():
            for s in range(segs_per_w):
                for cg in range(CG):
                    res_s[s, pl.ds(cg * _L, _L)] = zero
                    res_m[s, pl.ds(cg * _L, _L)] = zero

        base_out = pl.multiple_of(w * segs_per_w, 8)
        pltpu.sync_copy(res_s, sum_hbm.at[pl.ds(base_out, segs_per_w)])
        pltpu.sync_copy(res_m, max_hbm.at[pl.ds(base_out, segs_per_w)])

    return sc_kernel(x, batch)


def _tc_finish(sum_p, max_p, cnt_col, W1, b1, ln_g, ln_b, W2p, b2p):
    B, C = sum_p.shape

    def body(s_ref, m_ref, c_ref, W1_ref, b1_ref, g_ref, bb_ref,
             W2_ref, b2_ref, out_ref):
        s = s_ref[...]
        m = m_ref[...]
        c = c_ref[...]
        inv = 1.0 / jnp.maximum(c, 1.0)
        mean = s * inv
        h = (jnp.dot(mean, W1_ref[0:C, :], preferred_element_type=jnp.float32)
             + jnp.dot(m, W1_ref[C:2 * C, :], preferred_element_type=jnp.float32)
             + jnp.dot(s, W1_ref[2 * C:3 * C, :], preferred_element_type=jnp.float32)
             + b1_ref[...])
        mu = jnp.mean(h, axis=1, keepdims=True)
        var = jnp.mean((h - mu) * (h - mu), axis=1, keepdims=True)
        hn = (h - mu) * lax.rsqrt(var + 1e-5) * g_ref[...] + bb_ref[...]
        hr = jnp.maximum(hn, 0.0)
        logits = jnp.dot(hr, W2_ref[...], preferred_element_type=jnp.float32) + b2_ref[...]
        mx = jnp.max(logits, axis=1, keepdims=True)
        e = jnp.exp(logits - mx)
        wgt = e / jnp.sum(e, axis=1, keepdims=True)
        out_ref[...] = (wgt[:, 0:1] * mean + wgt[:, 1:2] * m + wgt[:, 2:3] * s)

    return pl.pallas_call(
        body,
        out_shape=jax.ShapeDtypeStruct((B, C), jnp.float32),
    )(sum_p, max_p, cnt_col, W1, b1, ln_g, ln_b, W2p, b2p)


def kernel(x, batch, W1, b1, ln_g, ln_b, W2, b2):
    N, C = x.shape
    H = W2.shape[1]

    G = -(-N // (_NS * _L)) * _L

    sum_p, max_p, off = _sc_segment_reduce(x, batch.astype(jnp.int32), G)
    cnt_col = (off[1:_B + 1] - off[:_B]).reshape(_B, 1).astype(jnp.float32)

    W2p = jnp.zeros((C, 128), jnp.float32).at[:, :H].set(W2)
    b2p = jnp.full((1, 128), -1e30, jnp.float32).at[0, :H].set(b2)
    return _tc_finish(sum_p, max_p, cnt_col, W1,
                      b1.reshape(1, C), ln_g.reshape(1, C),
                      ln_b.reshape(1, C), W2p, b2p)


# speedup vs baseline: 1.0161x; 1.0161x over previous
import functools

import jax
import jax.numpy as jnp
from jax import lax
from jax.experimental import pallas as pl
from jax.experimental.pallas import tpu as pltpu
from jax.experimental.pallas import tpu_sc as plsc

_NC = 2
_NS = 16
_L = 16
_NW = _NC * _NS
_B = 256
_K = 192
_OFFLEN = 272
_BIG = 0x7FFFFFFF


def _sc_segment_reduce(x, batch, G):
    N, C = x.shape
    CG = C // _L
    segs_per_w = _B // _NW
    OV = _OFFLEN // _L

    mesh = plsc.VectorSubcoreMesh(
        core_axis_name="c", subcore_axis_name="s",
        num_cores=_NC, num_subcores=_NS)

    @functools.partial(
        pl.kernel,
        out_type=(
            jax.ShapeDtypeStruct((_B, C), jnp.float32),
            jax.ShapeDtypeStruct((_B, C), jnp.float32),
            jax.ShapeDtypeStruct((_B, _L), jnp.float32),
        ),
        mesh=mesh,
        scratch_types=[
            pltpu.VMEM((G + _L,), jnp.int32),
            pltpu.VMEM((_OFFLEN,), jnp.int32),
            pltpu.VMEM((_NS, _OFFLEN), jnp.int32),
            pltpu.VMEM_SHARED((_NS, _OFFLEN), jnp.int32),
            pltpu.VMEM((2, _K, C), jnp.float32),
            pltpu.VMEM((segs_per_w, C), jnp.float32),
            pltpu.VMEM((segs_per_w, C), jnp.float32),
            pltpu.VMEM((segs_per_w, _L), jnp.float32),
            pltpu.SemaphoreType.DMA((2,)),
        ],
        compiler_params=pltpu.CompilerParams(needs_layout_passes=False),
    )
    def sc_kernel(x_hbm, b_hbm, sum_hbm, max_hbm, cnt_hbm,
                  idsbuf, offloc, comb, shared, buf_v, res_s, res_m, res_c,
                  sem):
        t = lax.axis_index("s")
        w = t * _NC + lax.axis_index("c")

        @pl.when(t == 0)
        def _():
            idsbuf[pl.ds(0, _L)] = jnp.full((_L,), -1, jnp.int32)
            pltpu.sync_copy(b_hbm.at[pl.ds(0, G + 8)],
                            idsbuf.at[pl.ds(8, G + 8)])

        @pl.when(jnp.logical_and(t > 0, t < _NS - 1))
        def _():
            pltpu.sync_copy(b_hbm.at[pl.ds(t * G - 8, G + _L)], idsbuf)

        @pl.when(t == _NS - 1)
        def _():
            pltpu.sync_copy(b_hbm.at[pl.ds((_NS - 1) * G - 8,
                                           N - (_NS - 1) * G + 8)],
                            idsbuf.at[pl.ds(0, N - (_NS - 1) * G + 8)])

        big = jnp.full((_L,), _BIG, jnp.int32)
        for v in range(OV):
            offloc[pl.ds(v * _L, _L)] = big

        iota = lax.iota(jnp.int32, _L)
        nv_last = -(-(N - (_NS - 1) * G) // _L)
        nvec = jnp.where(t == _NS - 1, nv_last, G // _L)

        def scan_body(k, _):
            i = _L * k
            cur = idsbuf[pl.ds(i + 8, _L)]
            prv = idsbuf[pl.ds(i + 7, _L)]
            pos = jnp.broadcast_to(t * G + i, (_L,)) + iota
            msk = jnp.logical_and(cur != prv, pos < N)
            plsc.store_scatter(offloc, [cur], pos, mask=msk)
            return 0

        lax.fori_loop(0, nvec, scan_body, 0)

        pltpu.sync_copy(offloc, shared.at[t])
        plsc.subcore_barrier()
        pltpu.sync_copy(shared, comb)

        carry = jnp.int32(_BIG)
        for v in reversed(range(OV)):
            mvec = comb[0, pl.ds(v * _L, _L)]
            for tt in range(1, _NS):
                mvec = jnp.minimum(mvec, comb[tt, pl.ds(v * _L, _L)])
            if v == OV - 1:
                mvec = jnp.minimum(mvec, jnp.full((_L,), N, jnp.int32))
            sm = -lax.rev(plsc.cummax(lax.rev(-mvec, (0,))), (0,))
            tot = jnp.minimum(sm, jnp.broadcast_to(carry, (_L,)))
            offloc[pl.ds(v * _L, _L)] = tot
            carry = tot[0]

        offv = offloc[pl.ds(w * segs_per_w, _L)]

        W0 = offv[0]
        W8 = offv[segs_per_w]
        base0 = pl.multiple_of((W0 // 8) * 8, 8)
        nblk = jnp.where(W8 > W0, (W8 - base0 + (_K - 1)) // _K, 0)

        def dma_for(blk, slot):
            base = base0 + blk * _K
            cb = pl.multiple_of(jnp.minimum(base, N - _K), 8)
            return pltpu.make_async_copy(
                x_hbm.at[pl.ds(cb, _K)], buf_v.at[slot], sem.at[slot])

        @pl.when(nblk > 0)
        def _():
            dma_for(0, 0).start()

        def acc_init():
            return (
                tuple(jnp.zeros((_L,), jnp.float32) for _ in range(CG)),
                tuple(jnp.full((_L,), -jnp.inf, jnp.float32) for _ in range(CG)),
            )

        zero = jnp.zeros((_L,), jnp.float32)

        def blk_body(blk, carry):
            p = lax.rem(blk, 2)
            dma_for(blk, p).wait()

            @pl.when(blk + 1 < nblk)
            def _():
                dma_for(blk + 1, 1 - p).start()

            base = base0 + blk * _K
            cb = pl.multiple_of(jnp.minimum(base, N - _K), 8)

            for s in range(segs_per_w):
                s_start = offv[s]
                s_end = offv[s + 1]
                lo = jnp.maximum(s_start, base) - cb
                hi = jnp.maximum(jnp.minimum(s_end, base + _K) - cb, lo)
                n2 = (hi - lo) // 2

                def pair_body(i, c, lo=lo, p=p):
                    sums, maxs = c
                    r = lo + 2 * i
                    new_s = []
                    new_m = []
                    for cg in range(CG):
                        v1 = buf_v[p, r, pl.ds(cg * _L, _L)]
                        v2 = buf_v[p, r + 1, pl.ds(cg * _L, _L)]
                        new_s.append(sums[cg] + (v1 + v2))
                        new_m.append(jnp.maximum(maxs[cg],
                                                 jnp.maximum(v1, v2)))
                    return (tuple(new_s), tuple(new_m))

                def row_body(r, c, p=p):
                    sums, maxs = c
                    new_s = []
                    new_m = []
                    for cg in range(CG):
                        v = buf_v[p, r, pl.ds(cg * _L, _L)]
                        new_s.append(sums[cg] + v)
                        new_m.append(jnp.maximum(maxs[cg], v))
                    return (tuple(new_s), tuple(new_m))

                carry = lax.fori_loop(0, n2, pair_body, carry)
                carry = lax.fori_loop(lo + 2 * n2, hi, row_body, carry)

                completes = jnp.logical_and(
                    s_end <= base + _K,
                    jnp.logical_or(s_end > base, blk == 0))

                def flush(c, s=s, s_start=s_start, s_end=s_end):
                    sums, maxs = c
                    res_c[s, pl.ds(0, _L)] = jnp.broadcast_to(
                        (s_end - s_start).astype(jnp.float32), (_L,))
                    for cg in range(CG):
                        res_s[s, pl.ds(cg * _L, _L)] = sums[cg]

                    @pl.when(s_end > s_start)
                    def _():
                        for cg in range(CG):
                            res_m[s, pl.ds(cg * _L, _L)] = maxs[cg]

                    @pl.when(s_end == s_start)
                    def _():
                        for cg in range(CG):
                            res_m[s, pl.ds(cg * _L, _L)] = zero

                    return acc_init()

                carry = lax.cond(completes, flush, lambda c: c, carry)

            return carry

        lax.fori_loop(0, nblk, blk_body, acc_init())

        @pl.when(nblk == 0)
        def _():
            for s in range(segs_per_w):
                res_c[s, pl.ds(0, _L)] = zero
                for cg in range(CG):
                    res_s[s, pl.ds(cg * _L, _L)] = zero
                    res_m[s, pl.ds(cg * _L, _L)] = zero

        base_out = pl.multiple_of(w * segs_per_w, 8)
        pltpu.sync_copy(res_s, sum_hbm.at[pl.ds(base_out, segs_per_w)])
        pltpu.sync_copy(res_m, max_hbm.at[pl.ds(base_out, segs_per_w)])
        pltpu.sync_copy(res_c, cnt_hbm.at[pl.ds(base_out, segs_per_w)])

    return sc_kernel(x, batch)


def _tc_finish(sum_p, max_p, cnt2, W1, b1, ln_g, ln_b, W2, b2):
    B, C = sum_p.shape

    def body(s_ref, m_ref, c_ref, W1_ref, b1_ref, g_ref, bb_ref,
             W2_ref, b2_ref, out_ref):
        s = s_ref[...]
        m = m_ref[...]
        c = c_ref[:, 0:1]
        inv = 1.0 / jnp.maximum(c, 1.0)
        mean = s * inv
        h = (jnp.dot(mean, W1_ref[0:C, :], preferred_element_type=jnp.float32)
             + jnp.dot(m, W1_ref[C:2 * C, :], preferred_element_type=jnp.float32)
             + jnp.dot(s, W1_ref[2 * C:3 * C, :], preferred_element_type=jnp.float32)
             + b1_ref[...])
        mu = jnp.mean(h, axis=1, keepdims=True)
        var = jnp.mean((h - mu) * (h - mu), axis=1, keepdims=True)
        hn = (h - mu) * lax.rsqrt(var + 1e-5) * g_ref[...] + bb_ref[...]
        hr = jnp.maximum(hn, 0.0)
        logits = jnp.dot(hr, W2_ref[...], preferred_element_type=jnp.float32) + b2_ref[...]
        mx = jnp.max(logits, axis=1, keepdims=True)
        e = jnp.exp(logits - mx)
        wgt = e / jnp.sum(e, axis=1, keepdims=True)
        out_ref[...] = (wgt[:, 0:1] * mean + wgt[:, 1:2] * m + wgt[:, 2:3] * s)

    return pl.pallas_call(
        body,
        out_shape=jax.ShapeDtypeStruct((B, C), jnp.float32),
    )(sum_p, max_p, cnt2, W1, b1, ln_g, ln_b, W2, b2)


def kernel(x, batch, W1, b1, ln_g, ln_b, W2, b2):
    N, C = x.shape
    H = W2.shape[1]

    G = -(-N // (_NS * _L)) * _L

    sum_p, max_p, cnt2 = _sc_segment_reduce(x, batch.astype(jnp.int32), G)

    return _tc_finish(sum_p, max_p, cnt2, W1,
                      b1.reshape(1, C), ln_g.reshape(1, C),
                      ln_b.reshape(1, C), W2, b2.reshape(1, H))
